# Initial kernel scaffold; baseline (speedup 1.0000x reference)
import jax
import jax.numpy as jnp
from jax.experimental import pallas as pl


def kernel(x, edge_index, W1, b1, W2, b2):
    raise NotImplementedError("write your pallas kernel here")


# speedup vs baseline: 13.8249x; 13.8249x over previous
import functools

import jax
import jax.numpy as jnp
from jax import lax
from jax.experimental import pallas as pl
from jax.experimental.pallas import tpu as pltpu
from jax.experimental.pallas import tpu_sc as plsc

N = 10000
E = 320000
D = 128
NP = 10240
NC = 2
NS = 16
NW = NC * NS
EPW = E // NW
C = 80
NCH = EPW // C
SLP = NP // NS

_MESH = plsc.VectorSubcoreMesh(core_axis_name="c", subcore_axis_name="s")
_SC_PARAMS = pltpu.CompilerParams(needs_layout_passes=False)


@functools.partial(
    pl.kernel,
    out_type=jax.ShapeDtypeStruct((NC, NP), jnp.float32),
    mesh=_MESH,
    compiler_params=_SC_PARAMS,
    scratch_types=[
        pltpu.VMEM((EPW,), jnp.int32),
        pltpu.VMEM((NP,), jnp.float32),
        pltpu.VMEM((NS, SLP), jnp.float32),
        pltpu.VMEM((SLP,), jnp.float32),
        pltpu.VMEM_SHARED((NS, NP), jnp.float32),
    ],
)
def _deg_hist(dst_hbm, out_hbm, dsts, hist, buf, comb, hist_all):
    cid = lax.axis_index("c")
    sid = lax.axis_index("s")
    wid = sid * NC + cid
    z16 = jnp.zeros((16,), jnp.float32)
    ones16 = jnp.ones((16,), jnp.float32)

    def zloop(i, _):
        hist[pl.ds(i * 16, 16)] = z16
        return 0

    lax.fori_loop(0, NP // 16, zloop, 0)
    pltpu.sync_copy(dst_hbm.at[pl.ds(wid * EPW, EPW)], dsts)

    def hloop(i, _):
        idx = dsts[pl.ds(i * 16, 16)]
        plsc.addupdate_scatter(hist, [idx], ones16)
        return 0

    lax.fori_loop(0, EPW // 16, hloop, 0)
    pltpu.sync_copy(hist, hist_all.at[sid])
    plsc.subcore_barrier()
    pltpu.sync_copy(hist_all.at[pl.ds(0, NS), pl.ds(sid * SLP, SLP)], buf)

    def cloop(k, _):
        v = buf[0, pl.ds(k * 16, 16)]
        for r in range(1, NS):
            v = v + buf[r, pl.ds(k * 16, 16)]
        comb[pl.ds(k * 16, 16)] = v
        return 0

    lax.fori_loop(0, SLP // 16, cloop, 0)
    pltpu.sync_copy(comb, out_hbm.at[cid, pl.ds(sid * SLP, SLP)])


@functools.partial(
    pl.kernel,
    out_type=jax.ShapeDtypeStruct((NC, NP, D), jnp.float32),
    mesh=_MESH,
    compiler_params=_SC_PARAMS,
    scratch_types=[
        pltpu.VMEM((C,), jnp.int32),
        pltpu.VMEM((C,), jnp.int32),
        pltpu.VMEM((C, D), jnp.float32),
        pltpu.VMEM_SHARED((NP, D), jnp.float32),
        pltpu.SemaphoreType.DMA,
    ],
)
def _edge_scatter(src_hbm, dst_hbm, g_hbm, out_hbm, sidx, didx, rows, acc_sh, sem):
    cid = lax.axis_index("c")
    sid = lax.axis_index("s")
    wid = sid * NC + cid
    z16 = jnp.zeros((16,), jnp.float32)

    def zloop(i, _):
        rows[i >> 3, pl.ds((i & 7) * 16, 16)] = z16
        return 0

    lax.fori_loop(0, C * D // 16, zloop, 0)
    for k in range(SLP // C):
        pltpu.sync_copy(rows, acc_sh.at[pl.ds(sid * SLP + k * C, C)])
    plsc.subcore_barrier()

    def eloop(j, _):
        off = wid * EPW + j * C
        pltpu.sync_copy(src_hbm.at[pl.ds(off, C)], sidx)
        pltpu.sync_copy(dst_hbm.at[pl.ds(off, C)], didx)
        pltpu.async_copy(g_hbm.at[sidx], rows, sem).wait()
        pltpu.sync_copy(rows, acc_sh.at[didx], add=True)
        return 0

    lax.fori_loop(0, NCH, eloop, 0)
    plsc.subcore_barrier()
    pltpu.sync_copy(
        acc_sh.at[pl.ds(sid * SLP, SLP)],
        out_hbm.at[cid, pl.ds(sid * SLP, SLP)],
    )


BR = 1280
GRID = NP // BR

_row_spec = pl.BlockSpec((BR, D), lambda i: (i, 0))
_col_spec = pl.BlockSpec((BR, 1), lambda i: (i, 0))
_w_spec = pl.BlockSpec((D, D), lambda i: (0, 0))
_b_spec = pl.BlockSpec((1, D), lambda i: (0, 0))


def _scale_matmul_body(d0_ref, d1_ref, x_ref, w_ref, g_ref):
    s = lax.rsqrt(d0_ref[...] + d1_ref[...] + 1.0)
    g_ref[...] = jnp.dot(x_ref[...], w_ref[...],
                         preferred_element_type=jnp.float32) * s


def _scale_matmul(d0, d1, x, w):
    return pl.pallas_call(
        _scale_matmul_body,
        out_shape=jax.ShapeDtypeStruct((NP, D), jnp.float32),
        grid=(GRID,),
        in_specs=[_col_spec, _col_spec, _row_spec, _w_spec],
        out_specs=_row_spec,
    )(d0, d1, x, w)


def _mid_body(d0_ref, d1_ref, a0_ref, a1_ref, g_ref, b_ref, w_ref, o_ref):
    s = lax.rsqrt(d0_ref[...] + d1_ref[...] + 1.0)
    pre = s * (a0_ref[...] + a1_ref[...] + g_ref[...]) + b_ref[...]
    h = jnp.maximum(pre, 0.0)
    o_ref[...] = jnp.dot(h, w_ref[...], preferred_element_type=jnp.float32) * s


def _mid(d0, d1, a0, a1, g, b, w):
    return pl.pallas_call(
        _mid_body,
        out_shape=jax.ShapeDtypeStruct((NP, D), jnp.float32),
        grid=(GRID,),
        in_specs=[_col_spec, _col_spec, _row_spec, _row_spec, _row_spec,
                  _b_spec, _w_spec],
        out_specs=_row_spec,
    )(d0, d1, a0, a1, g, b, w)


def _final_body(d0_ref, d1_ref, a0_ref, a1_ref, g_ref, b_ref, o_ref):
    s = lax.rsqrt(d0_ref[...] + d1_ref[...] + 1.0)
    o_ref[...] = s * (a0_ref[...] + a1_ref[...] + g_ref[...]) + b_ref[...]


def _final(d0, d1, a0, a1, g, b):
    return pl.pallas_call(
        _final_body,
        out_shape=jax.ShapeDtypeStruct((NP, D), jnp.float32),
        grid=(GRID,),
        in_specs=[_col_spec, _col_spec, _row_spec, _row_spec, _row_spec,
                  _b_spec],
        out_specs=_row_spec,
    )(d0, d1, a0, a1, g, b)


def kernel(x, edge_index, W1, b1, W2, b2):
    src = edge_index[0]
    dst = edge_index[1]
    x_pad = jnp.pad(x, ((0, NP - N), (0, 0)))
    b1r = b1.reshape(1, D)
    b2r = b2.reshape(1, D)

    deg2 = _deg_hist(dst)
    d0 = deg2[0].reshape(NP, 1)
    d1 = deg2[1].reshape(NP, 1)

    g1 = _scale_matmul(d0, d1, x_pad, W1)
    acc1 = _edge_scatter(src, dst, g1)
    g2 = _mid(d0, d1, acc1[0], acc1[1], g1, b1r, W2)
    acc2 = _edge_scatter(src, dst, g2)
    out = _final(d0, d1, acc2[0], acc2[1], g2, b2r)
    return out[:N]


# speedup vs baseline: 27.4344x; 1.9844x over previous
import functools

import jax
import jax.numpy as jnp
from jax import lax
from jax.experimental import pallas as pl
from jax.experimental.pallas import tpu as pltpu
from jax.experimental.pallas import tpu_sc as plsc

N = 10000
E = 320000
D = 128
NP = 10240
NC = 2
NS = 16
NW = NC * NS
EPW = E // NW
CH = 125
NSUP = EPW // CH
SLP = NP // NS
NIB = 3

_MESH = plsc.VectorSubcoreMesh(core_axis_name="c", subcore_axis_name="s")
_SC_PARAMS = pltpu.CompilerParams(needs_layout_passes=False)


@functools.partial(
    pl.kernel,
    out_type=jax.ShapeDtypeStruct((NC, NP), jnp.float32),
    mesh=_MESH,
    compiler_params=_SC_PARAMS,
    scratch_types=[
        pltpu.VMEM((EPW,), jnp.int32),
        pltpu.VMEM((NP,), jnp.float32),
        pltpu.VMEM((NS, SLP), jnp.float32),
        pltpu.VMEM((SLP,), jnp.float32),
        pltpu.VMEM_SHARED((NS, NP), jnp.float32),
    ],
)
def _deg_hist(dst_hbm, out_hbm, dsts, hist, buf, comb, hist_all):
    cid = lax.axis_index("c")
    sid = lax.axis_index("s")
    wid = sid * NC + cid
    z16 = jnp.zeros((16,), jnp.float32)
    ones16 = jnp.ones((16,), jnp.float32)

    def zloop(i, _):
        hist[pl.ds(i * 16, 16)] = z16
        return 0

    lax.fori_loop(0, NP // 16, zloop, 0)
    pltpu.sync_copy(dst_hbm.at[pl.ds(wid * EPW, EPW)], dsts)

    def hloop(i, _):
        idx = dsts[pl.ds(i * 16, 16)]
        plsc.addupdate_scatter(hist, [idx], ones16)
        return 0

    lax.fori_loop(0, EPW // 16, hloop, 0)
    pltpu.sync_copy(hist, hist_all.at[sid])
    plsc.subcore_barrier()
    pltpu.sync_copy(hist_all.at[pl.ds(0, NS), pl.ds(sid * SLP, SLP)], buf)

    def cloop(k, _):
        v = buf[0, pl.ds(k * 16, 16)]
        for r in range(1, NS):
            v = v + buf[r, pl.ds(k * 16, 16)]
        comb[pl.ds(k * 16, 16)] = v
        return 0

    lax.fori_loop(0, SLP // 16, cloop, 0)
    pltpu.sync_copy(comb, out_hbm.at[cid, pl.ds(sid * SLP, SLP)])


@functools.partial(
    pl.kernel,
    out_type=jax.ShapeDtypeStruct((NC, NP, D), jnp.float32),
    mesh=_MESH,
    compiler_params=_SC_PARAMS,
    scratch_types=[
        pltpu.VMEM((NIB, CH), jnp.int32),
        pltpu.VMEM((NIB, CH), jnp.int32),
        pltpu.VMEM((2, CH, D), jnp.float32),
        pltpu.VMEM_SHARED((NP, D), jnp.float32),
        pltpu.SemaphoreType.DMA,
        pltpu.SemaphoreType.DMA,
        pltpu.SemaphoreType.DMA,
    ],
)
def _edge_scatter(src_hbm, dst_hbm, g_hbm, z_hbm, out_hbm, sidx, didx, rows,
                  acc_sh, gsem, ssem, isem):
    cid = lax.axis_index("c")
    sid = lax.axis_index("s")
    wid = sid * NC + cid
    base = sid * SLP

    def fire_g(slot, buf):
        pltpu.async_copy(g_hbm.at[sidx.at[slot]], rows.at[buf], gsem)

    def drain_g(slot, buf):
        pltpu.make_async_copy(g_hbm.at[sidx.at[slot]], rows.at[buf],
                              gsem).wait()

    def fire_s(slot, buf):
        pltpu.async_copy(rows.at[buf], acc_sh.at[didx.at[slot]], ssem,
                         add=True)

    def drain_s(slot, buf):
        pltpu.make_async_copy(rows.at[buf], acc_sh.at[didx.at[slot]],
                              ssem).wait()

    def fire_idx(m, slot):
        pltpu.async_copy(src_hbm.at[wid, m], sidx.at[slot], isem)
        pltpu.async_copy(dst_hbm.at[wid, m], didx.at[slot], isem)

    def drain_idx(m, slot):
        pltpu.make_async_copy(src_hbm.at[wid, m], sidx.at[slot], isem).wait()
        pltpu.make_async_copy(dst_hbm.at[wid, m], didx.at[slot], isem).wait()

    pltpu.sync_copy(src_hbm.at[wid, 0], sidx.at[0])
    pltpu.sync_copy(dst_hbm.at[wid, 0], didx.at[0])
    fire_idx(1, 1)
    fire_g(0, 0)
    pltpu.sync_copy(z_hbm, acc_sh.at[pl.ds(base, SLP)])
    plsc.subcore_barrier()

    def step(m, _):
        p = lax.rem(m, NIB)
        p1 = lax.rem(m + 1, NIB)
        p2 = lax.rem(m + 2, NIB)
        r = lax.rem(m, 2)
        r1 = lax.rem(m + 1, 2)
        drain_g(p, r)
        fire_s(p, r)
        drain_idx(m + 1, p1)
        fire_idx(m + 2, p2)
        fire_g(p1, r1)
        drain_s(p, r)
        return 0

    lax.fori_loop(0, NSUP - 2, step, 0)

    m = NSUP - 2
    p, p1, r, r1 = m % NIB, (m + 1) % NIB, m % 2, (m + 1) % 2
    drain_g(p, r)
    fire_s(p, r)
    drain_idx(m + 1, p1)
    fire_g(p1, r1)
    drain_s(p, r)

    m = NSUP - 1
    p, r = m % NIB, m % 2
    drain_g(p, r)
    fire_s(p, r)
    drain_s(p, r)

    plsc.subcore_barrier()
    pltpu.sync_copy(
        acc_sh.at[pl.ds(base, SLP)],
        out_hbm.at[cid, pl.ds(base, SLP)],
    )


BR = 1280
GRID = NP // BR

_row_spec = pl.BlockSpec((BR, D), lambda i: (i, 0))
_col_spec = pl.BlockSpec((BR, 1), lambda i: (i, 0))
_w_spec = pl.BlockSpec((D, D), lambda i: (0, 0))
_b_spec = pl.BlockSpec((1, D), lambda i: (0, 0))


def _scale_matmul_body(d0_ref, d1_ref, x_ref, w_ref, g_ref):
    s = lax.rsqrt(d0_ref[...] + d1_ref[...] + 1.0)
    g_ref[...] = jnp.dot(x_ref[...], w_ref[...],
                         preferred_element_type=jnp.float32) * s


def _scale_matmul(d0, d1, x, w):
    return pl.pallas_call(
        _scale_matmul_body,
        out_shape=jax.ShapeDtypeStruct((NP, D), jnp.float32),
        grid=(GRID,),
        in_specs=[_col_spec, _col_spec, _row_spec, _w_spec],
        out_specs=_row_spec,
    )(d0, d1, x, w)


def _mid_body(d0_ref, d1_ref, a0_ref, a1_ref, g_ref, b_ref, w_ref, o_ref):
    s = lax.rsqrt(d0_ref[...] + d1_ref[...] + 1.0)
    pre = s * (a0_ref[...] + a1_ref[...] + g_ref[...]) + b_ref[...]
    h = jnp.maximum(pre, 0.0)
    o_ref[...] = jnp.dot(h, w_ref[...], preferred_element_type=jnp.float32) * s


def _mid(d0, d1, a0, a1, g, b, w):
    return pl.pallas_call(
        _mid_body,
        out_shape=jax.ShapeDtypeStruct((NP, D), jnp.float32),
        grid=(GRID,),
        in_specs=[_col_spec, _col_spec, _row_spec, _row_spec, _row_spec,
                  _b_spec, _w_spec],
        out_specs=_row_spec,
    )(d0, d1, a0, a1, g, b, w)


def _final_body(d0_ref, d1_ref, a0_ref, a1_ref, g_ref, b_ref, o_ref):
    s = lax.rsqrt(d0_ref[...] + d1_ref[...] + 1.0)
    o_ref[...] = s * (a0_ref[...] + a1_ref[...] + g_ref[...]) + b_ref[...]


def _final(d0, d1, a0, a1, g, b):
    return pl.pallas_call(
        _final_body,
        out_shape=jax.ShapeDtypeStruct((NP, D), jnp.float32),
        grid=(GRID,),
        in_specs=[_col_spec, _col_spec, _row_spec, _row_spec, _row_spec,
                  _b_spec],
        out_specs=_row_spec,
    )(d0, d1, a0, a1, g, b)


def kernel(x, edge_index, W1, b1, W2, b2):
    src = edge_index[0].reshape(NW, NSUP, CH)
    dst_flat = edge_index[1]
    dst = dst_flat.reshape(NW, NSUP, CH)
    zrows = jnp.zeros((SLP, D), jnp.float32)
    x_pad = jnp.pad(x, ((0, NP - N), (0, 0)))
    b1r = b1.reshape(1, D)
    b2r = b2.reshape(1, D)

    deg2 = _deg_hist(dst_flat)
    d0 = deg2[0].reshape(NP, 1)
    d1 = deg2[1].reshape(NP, 1)

    g1 = _scale_matmul(d0, d1, x_pad, W1)
    acc1 = _edge_scatter(src, dst, g1, zrows)
    g2 = _mid(d0, d1, acc1[0], acc1[1], g1, b1r, W2)
    acc2 = _edge_scatter(src, dst, g2, zrows)
    out = _final(d0, d1, acc2[0], acc2[1], g2, b2r)
    return out[:N]
